# Initial kernel scaffold; baseline (speedup 1.0000x reference)
import jax
import jax.numpy as jnp
from jax.experimental import pallas as pl


def kernel(x, edge_index, edge_attr, edge_type, w1, q1, k1, le1, e1, b1, w2, q2, k2, le2, e2, b2):
    raise NotImplementedError("write your pallas kernel here")


# speedup vs baseline: 42.9316x; 42.9316x over previous
import functools
import jax
import jax.numpy as jnp
from jax import lax
from jax.experimental import pallas as pl
from jax.experimental.pallas import tpu as pltpu
from jax.experimental.pallas import tpu_sc as plsc

N = 10000
E = 320000
F = 128
R = 8
H = 4
O = 32
NB = 1000
EB = 2000
NCORE = 2
NSUB = 16
NW = NCORE * NSUB
EPW = E // NW
C = 80
NCHUNK = EPW // C
NP = 10240
RPS = NP // NSUB


def _prep_body(x_ref, w_ref, qp_ref, kp_ref, xw_ref, xwk_ref, xwq_ref):
    xw = jnp.dot(x_ref[...], w_ref[0], preferred_element_type=jnp.float32)
    xw_ref[...] = xw
    xwk_ref[...] = jnp.dot(xw, kp_ref[...], preferred_element_type=jnp.float32)
    xwq_ref[...] = jnp.dot(xw, qp_ref[...], preferred_element_type=jnp.float32)


def _prep(x, w, qpad, kpad):
    return pl.pallas_call(
        _prep_body,
        grid=(R, N // NB),
        in_specs=[
            pl.BlockSpec((NB, F), lambda r, i: (i, 0)),
            pl.BlockSpec((1, F, F), lambda r, i: (r, 0, 0)),
            pl.BlockSpec((F, 16), lambda r, i: (0, 0)),
            pl.BlockSpec((F, 16), lambda r, i: (0, 0)),
        ],
        out_specs=[
            pl.BlockSpec((NB, F), lambda r, i: (r * (N // NB) + i, 0)),
            pl.BlockSpec((NB, 16), lambda r, i: (r * (N // NB) + i, 0)),
            pl.BlockSpec((NB, 16), lambda r, i: (r * (N // NB) + i, 0)),
        ],
        out_shape=[
            jax.ShapeDtypeStruct((R * N, F), jnp.float32),
            jax.ShapeDtypeStruct((R * N, 16), jnp.float32),
            jax.ShapeDtypeStruct((R * N, 16), jnp.float32),
        ],
    )(x, w, qpad, kpad)


def _ae_body(ea_ref, le1_ref, e1_ref, le2_ref, e2_ref, ae1_ref, ae2_ref):
    ea = ea_ref[...]
    t1 = jnp.dot(ea, le1_ref[...], preferred_element_type=jnp.float32)
    ae1_ref[...] = jnp.dot(t1, e1_ref[...], preferred_element_type=jnp.float32)
    t2 = jnp.dot(ea, le2_ref[...], preferred_element_type=jnp.float32)
    ae2_ref[...] = jnp.dot(t2, e2_ref[...], preferred_element_type=jnp.float32)


def _ae(edge_attr, le1, e1pad, le2, e2pad):
    return pl.pallas_call(
        _ae_body,
        grid=(E // EB,),
        in_specs=[
            pl.BlockSpec((EB, 16), lambda i: (i, 0)),
            pl.BlockSpec((16, F), lambda i: (0, 0)),
            pl.BlockSpec((F, 16), lambda i: (0, 0)),
            pl.BlockSpec((16, F), lambda i: (0, 0)),
            pl.BlockSpec((F, 16), lambda i: (0, 0)),
        ],
        out_specs=[
            pl.BlockSpec((EB, 16), lambda i: (i, 0)),
            pl.BlockSpec((EB, 16), lambda i: (i, 0)),
        ],
        out_shape=[
            jax.ShapeDtypeStruct((E, 16), jnp.float32),
            jax.ShapeDtypeStruct((E, 16), jnp.float32),
        ],
    )(edge_attr, le1, e1pad, le2, e2pad)


def _edge_body(src_h, dst_h, rel_h, ae_h, xw_h, xwk_h, xwq_h, zm_h, zd_h,
               msum_h, dsum_h,
               srcv, dstv, relv, fiv, fjv, aev, xwv, xwkv, xwqv, exv,
               accm, accd, sem):
    c = lax.axis_index("c")
    s = lax.axis_index("s")
    base = (c * NSUB + s) * EPW

    pltpu.sync_copy(zm_h.at[pl.ds(s * RPS, RPS)], accm.at[pl.ds(s * RPS, RPS)])
    pltpu.sync_copy(zd_h.at[pl.ds(s * RPS, RPS)], accd.at[pl.ds(s * RPS, RPS)])
    plsc.subcore_barrier()

    def chunk(i, carry):
        off = base + i * C
        pltpu.sync_copy(src_h.at[pl.ds(off, C)], srcv)
        pltpu.sync_copy(dst_h.at[pl.ds(off, C)], dstv)
        pltpu.sync_copy(rel_h.at[pl.ds(off, C)], relv)
        pltpu.sync_copy(ae_h.at[pl.ds(off, C)], aev)
        for g in range(C // 16):
            sl = pl.ds(g * 16, 16)
            r16 = relv[sl] * N
            fiv[sl] = r16 + dstv[sl]
            fjv[sl] = r16 + srcv[sl]
        pltpu.sync_copy(xw_h.at[fjv], xwv)
        pltpu.sync_copy(xwk_h.at[fjv], xwkv)
        pltpu.sync_copy(xwq_h.at[fiv], xwqv)
        for e in range(C):
            a = xwqv[e, :] + xwkv[e, :] + aev[e, :]
            a = jnp.maximum(a, 0.2 * a)
            exr = jnp.exp(a)
            exv[e, :] = exr
            for h in range(H):
                sc = exr[h]
                for t in range(O // 16):
                    sl = pl.ds(h * O + t * 16, 16)
                    xwv[e, sl] = xwv[e, sl] * sc
        pltpu.sync_copy(xwv, accm.at[dstv], add=True)
        pltpu.sync_copy(exv, accd.at[dstv], add=True)
        return carry

    lax.fori_loop(0, NCHUNK, chunk, 0)

    plsc.subcore_barrier()
    pltpu.sync_copy(accm.at[pl.ds(s * RPS, RPS)],
                    msum_h.at[c, pl.ds(s * RPS, RPS)])
    pltpu.sync_copy(accd.at[pl.ds(s * RPS, RPS)],
                    dsum_h.at[c, pl.ds(s * RPS, RPS)])


@functools.partial(
    pl.kernel,
    out_type=[
        jax.ShapeDtypeStruct((NCORE, NP, F), jnp.float32),
        jax.ShapeDtypeStruct((NCORE, NP, 16), jnp.float32),
    ],
    mesh=plsc.VectorSubcoreMesh(core_axis_name="c", subcore_axis_name="s"),
    scratch_types=[
        pltpu.VMEM((C,), jnp.int32),
        pltpu.VMEM((C,), jnp.int32),
        pltpu.VMEM((C,), jnp.int32),
        pltpu.VMEM((C,), jnp.int32),
        pltpu.VMEM((C,), jnp.int32),
        pltpu.VMEM((C, 16), jnp.float32),
        pltpu.VMEM((C, F), jnp.float32),
        pltpu.VMEM((C, 16), jnp.float32),
        pltpu.VMEM((C, 16), jnp.float32),
        pltpu.VMEM((C, 16), jnp.float32),
        pltpu.VMEM_SHARED((NP, F), jnp.float32),
        pltpu.VMEM_SHARED((NP, 16), jnp.float32),
        pltpu.SemaphoreType.DMA,
    ],
    compiler_params=pltpu.CompilerParams(use_tc_tiling_on_sc=False),
)
def _edge_pass(*args):
    _edge_body(*args)


def _comb1_body(m_ref, d_ref, sel_ref, b_ref, h_ref):
    M = m_ref[0] + m_ref[1]
    D = d_ref[0] + d_ref[1]
    dm = jnp.dot(D, sel_ref[...], preferred_element_type=jnp.float32)
    h_ref[...] = jnp.maximum(M / (dm + 1e-16) + b_ref[...], 0.0)


def _comb1(msum, dsum, sel, brow):
    return pl.pallas_call(
        _comb1_body,
        grid=(N // NB,),
        in_specs=[
            pl.BlockSpec((2, NB, F), lambda i: (0, i, 0)),
            pl.BlockSpec((2, NB, 16), lambda i: (0, i, 0)),
            pl.BlockSpec((16, F), lambda i: (0, 0)),
            pl.BlockSpec((1, F), lambda i: (0, 0)),
        ],
        out_specs=pl.BlockSpec((NB, F), lambda i: (i, 0)),
        out_shape=jax.ShapeDtypeStruct((N, F), jnp.float32),
    )(msum, dsum, sel, brow)


def _comb2_body(m_ref, d_ref, sel_ref, t_ref, b_ref, o_ref):
    M = m_ref[0] + m_ref[1]
    D = d_ref[0] + d_ref[1]
    dm = jnp.dot(D, sel_ref[...], preferred_element_type=jnp.float32)
    val = M / (dm + 1e-16)
    o_ref[...] = jnp.dot(val, t_ref[...], preferred_element_type=jnp.float32) + b_ref[...]


def _comb2(msum, dsum, sel, tmat, brow):
    return pl.pallas_call(
        _comb2_body,
        grid=(N // NB,),
        in_specs=[
            pl.BlockSpec((2, NB, F), lambda i: (0, i, 0)),
            pl.BlockSpec((2, NB, 16), lambda i: (0, i, 0)),
            pl.BlockSpec((16, F), lambda i: (0, 0)),
            pl.BlockSpec((F, O), lambda i: (0, 0)),
            pl.BlockSpec((1, O), lambda i: (0, 0)),
        ],
        out_specs=pl.BlockSpec((NB, O), lambda i: (i, 0)),
        out_shape=jax.ShapeDtypeStruct((N, O), jnp.float32),
    )(msum, dsum, sel, tmat, brow)


def kernel(x, edge_index, edge_attr, edge_type, w1, q1, k1, le1, e1, b1,
           w2, q2, k2, le2, e2, b2):
    src = edge_index[0]
    dst = edge_index[1]

    pad12 = ((0, 0), (0, 12))
    q1p = jnp.pad(q1, pad12)
    k1p = jnp.pad(k1, pad12)
    q2p = jnp.pad(q2, pad12)
    k2p = jnp.pad(k2, pad12)
    e1p = jnp.pad(e1, pad12)
    e2p = jnp.pad(e2, pad12)

    hh = jnp.arange(16)[:, None]
    cc = jnp.arange(F)[None, :]
    sel = (cc // O == hh).astype(jnp.float32)
    tmat = ((jnp.arange(F)[:, None] % O) == jnp.arange(O)[None, :]).astype(
        jnp.float32) / H

    zm = jnp.zeros((NP, F), jnp.float32)
    zd = jnp.zeros((NP, 16), jnp.float32)

    ae1, ae2 = _ae(edge_attr, le1, e1p, le2, e2p)

    xw1, xwk1, xwq1 = _prep(x, w1, q1p, k1p)
    m1, d1 = _edge_pass(src, dst, edge_type, ae1, xw1, xwk1, xwq1, zm, zd)
    h = _comb1(m1, d1, sel, b1.reshape(1, F))

    xw2, xwk2, xwq2 = _prep(h, w2, q2p, k2p)
    m2, d2 = _edge_pass(src, dst, edge_type, ae2, xw2, xwk2, xwq2, zm, zd)
    out = _comb2(m2, d2, sel, tmat, b2.reshape(1, O))
    return out


# speedup vs baseline: 47.8828x; 1.1153x over previous
import functools
import jax
import jax.numpy as jnp
from jax import lax
from jax.experimental import pallas as pl
from jax.experimental.pallas import tpu as pltpu
from jax.experimental.pallas import tpu_sc as plsc

N = 10000
E = 320000
F = 128
R = 8
H = 4
O = 32
NB = 1000
EB = 2000
NCORE = 2
NSUB = 16
NW = NCORE * NSUB
EPW = E // NW
SUB = 80
NSUBC = 1
C = SUB * NSUBC
NCHUNK = EPW // C
FJ = F + 16
NP = 10240
RPS = NP // NSUB


def _prep_body(x_ref, w_ref, qp_ref, kp_ref, xwj_ref, xwq_ref):
    xw = jnp.dot(x_ref[...], w_ref[0], preferred_element_type=jnp.float32)
    xwj_ref[:, :F] = xw
    xwj_ref[:, F:] = jnp.dot(xw, kp_ref[...], preferred_element_type=jnp.float32)
    xwq_ref[...] = jnp.dot(xw, qp_ref[...], preferred_element_type=jnp.float32)


def _prep(x, w, qpad, kpad):
    return pl.pallas_call(
        _prep_body,
        grid=(R, N // NB),
        in_specs=[
            pl.BlockSpec((NB, F), lambda r, i: (i, 0)),
            pl.BlockSpec((1, F, F), lambda r, i: (r, 0, 0)),
            pl.BlockSpec((F, 16), lambda r, i: (0, 0)),
            pl.BlockSpec((F, 16), lambda r, i: (0, 0)),
        ],
        out_specs=[
            pl.BlockSpec((NB, FJ), lambda r, i: (r * (N // NB) + i, 0)),
            pl.BlockSpec((NB, 16), lambda r, i: (r * (N // NB) + i, 0)),
        ],
        out_shape=[
            jax.ShapeDtypeStruct((R * N, FJ), jnp.float32),
            jax.ShapeDtypeStruct((R * N, 16), jnp.float32),
        ],
    )(x, w, qpad, kpad)


def _ae_body(ea_ref, le1_ref, e1_ref, le2_ref, e2_ref, ae1_ref, ae2_ref):
    ea = ea_ref[...]
    t1 = jnp.dot(ea, le1_ref[...], preferred_element_type=jnp.float32)
    ae1_ref[...] = jnp.dot(t1, e1_ref[...], preferred_element_type=jnp.float32)
    t2 = jnp.dot(ea, le2_ref[...], preferred_element_type=jnp.float32)
    ae2_ref[...] = jnp.dot(t2, e2_ref[...], preferred_element_type=jnp.float32)


def _ae(edge_attr, le1, e1pad, le2, e2pad):
    return pl.pallas_call(
        _ae_body,
        grid=(E // EB,),
        in_specs=[
            pl.BlockSpec((EB, 16), lambda i: (i, 0)),
            pl.BlockSpec((16, F), lambda i: (0, 0)),
            pl.BlockSpec((F, 16), lambda i: (0, 0)),
            pl.BlockSpec((16, F), lambda i: (0, 0)),
            pl.BlockSpec((F, 16), lambda i: (0, 0)),
        ],
        out_specs=[
            pl.BlockSpec((EB, 16), lambda i: (i, 0)),
            pl.BlockSpec((EB, 16), lambda i: (i, 0)),
        ],
        out_shape=[
            jax.ShapeDtypeStruct((E, 16), jnp.float32),
            jax.ShapeDtypeStruct((E, 16), jnp.float32),
        ],
    )(edge_attr, le1, e1pad, le2, e2pad)


def _edge_body(src_h, dst_h, rel_h, ae_h, xwj_h, xwq_h, zm_h, msum_h,
               srcv, dstv, relv, fiv, fjv, aev, xwv, xwqv, accm, sem):
    c = lax.axis_index("c")
    s = lax.axis_index("s")
    base = (c * NSUB + s) * EPW

    pltpu.sync_copy(zm_h.at[pl.ds(s * RPS, RPS)], accm.at[pl.ds(s * RPS, RPS)])
    plsc.subcore_barrier()

    def chunk(i, carry):
        off = base + i * C
        cps = []
        for j in range(NSUBC):
            o = off + j * SUB
            cps.append(pltpu.async_copy(src_h.at[pl.ds(o, SUB)], srcv.at[j], sem))
            cps.append(pltpu.async_copy(dst_h.at[pl.ds(o, SUB)], dstv.at[j], sem))
            cps.append(pltpu.async_copy(rel_h.at[pl.ds(o, SUB)], relv.at[j], sem))
        cps.append(pltpu.async_copy(ae_h.at[pl.ds(off, C)], aev, sem))
        for cp in cps:
            cp.wait()

        def idxbody(j, cr):
            for g in range(SUB // 16):
                sl = pl.ds(g * 16, 16)
                r16 = relv[j, sl] * N
                fiv[j, sl] = r16 + dstv[j, sl]
                fjv[j, sl] = r16 + srcv[j, sl]
            return cr
        lax.fori_loop(0, NSUBC, idxbody, 0)

        cps = []
        for j in range(NSUBC):
            sl = pl.ds(j * SUB, SUB)
            cps.append(pltpu.async_copy(xwj_h.at[fjv.at[j]], xwv.at[sl], sem))
            cps.append(pltpu.async_copy(xwq_h.at[fiv.at[j]], xwqv.at[sl], sem))
        for cp in cps:
            cp.wait()

        def cbody(g, cr):
            rb = g * 16
            for e in range(16):
                row = rb + e
                a = xwqv[row, :] + xwv[row, pl.ds(F, 16)] + aev[row, :]
                a = jnp.maximum(a, 0.2 * a)
                exr = jnp.exp(a)
                xwv[row, pl.ds(F, 16)] = exr
                for h in range(H):
                    sc = exr[h]
                    for t in range(O // 16):
                        sl = pl.ds(h * O + t * 16, 16)
                        xwv[row, sl] = xwv[row, sl] * sc
            return cr
        lax.fori_loop(0, C // 16, cbody, 0)

        cps = []
        for j in range(NSUBC):
            sl = pl.ds(j * SUB, SUB)
            cps.append(pltpu.async_copy(xwv.at[sl], accm.at[dstv.at[j]], sem,
                                        add=True))
        for cp in cps:
            cp.wait()
        return carry

    lax.fori_loop(0, NCHUNK, chunk, 0)

    plsc.subcore_barrier()
    pltpu.sync_copy(accm.at[pl.ds(s * RPS, RPS)],
                    msum_h.at[c, pl.ds(s * RPS, RPS)])


@functools.partial(
    pl.kernel,
    out_type=jax.ShapeDtypeStruct((NCORE, NP, FJ), jnp.float32),
    mesh=plsc.VectorSubcoreMesh(core_axis_name="c", subcore_axis_name="s"),
    scratch_types=[
        pltpu.VMEM((NSUBC, SUB), jnp.int32),
        pltpu.VMEM((NSUBC, SUB), jnp.int32),
        pltpu.VMEM((NSUBC, SUB), jnp.int32),
        pltpu.VMEM((NSUBC, SUB), jnp.int32),
        pltpu.VMEM((NSUBC, SUB), jnp.int32),
        pltpu.VMEM((C, 16), jnp.float32),
        pltpu.VMEM((C, FJ), jnp.float32),
        pltpu.VMEM((C, 16), jnp.float32),
        pltpu.VMEM_SHARED((NP, FJ), jnp.float32),
        pltpu.SemaphoreType.DMA,
    ],
    compiler_params=pltpu.CompilerParams(use_tc_tiling_on_sc=False),
)
def _edge_pass(*args):
    _edge_body(*args)


def _comb1_body(m_ref, sel_ref, b_ref, h_ref):
    ms = m_ref[0] + m_ref[1]
    M = ms[:, :F]
    D = ms[:, F:]
    dm = jnp.dot(D, sel_ref[...], preferred_element_type=jnp.float32)
    h_ref[...] = jnp.maximum(M / (dm + 1e-16) + b_ref[...], 0.0)


def _comb1(msum, sel, brow):
    return pl.pallas_call(
        _comb1_body,
        grid=(N // NB,),
        in_specs=[
            pl.BlockSpec((2, NB, FJ), lambda i: (0, i, 0)),
            pl.BlockSpec((16, F), lambda i: (0, 0)),
            pl.BlockSpec((1, F), lambda i: (0, 0)),
        ],
        out_specs=pl.BlockSpec((NB, F), lambda i: (i, 0)),
        out_shape=jax.ShapeDtypeStruct((N, F), jnp.float32),
    )(msum, sel, brow)


def _comb2_body(m_ref, sel_ref, t_ref, b_ref, o_ref):
    ms = m_ref[0] + m_ref[1]
    M = ms[:, :F]
    D = ms[:, F:]
    dm = jnp.dot(D, sel_ref[...], preferred_element_type=jnp.float32)
    val = M / (dm + 1e-16)
    o_ref[...] = jnp.dot(val, t_ref[...], preferred_element_type=jnp.float32) + b_ref[...]


def _comb2(msum, sel, tmat, brow):
    return pl.pallas_call(
        _comb2_body,
        grid=(N // NB,),
        in_specs=[
            pl.BlockSpec((2, NB, FJ), lambda i: (0, i, 0)),
            pl.BlockSpec((16, F), lambda i: (0, 0)),
            pl.BlockSpec((F, O), lambda i: (0, 0)),
            pl.BlockSpec((1, O), lambda i: (0, 0)),
        ],
        out_specs=pl.BlockSpec((NB, O), lambda i: (i, 0)),
        out_shape=jax.ShapeDtypeStruct((N, O), jnp.float32),
    )(msum, sel, tmat, brow)


def kernel(x, edge_index, edge_attr, edge_type, w1, q1, k1, le1, e1, b1,
           w2, q2, k2, le2, e2, b2):
    src = edge_index[0]
    dst = edge_index[1]

    pad12 = ((0, 0), (0, 12))
    q1p = jnp.pad(q1, pad12)
    k1p = jnp.pad(k1, pad12)
    q2p = jnp.pad(q2, pad12)
    k2p = jnp.pad(k2, pad12)
    e1p = jnp.pad(e1, pad12)
    e2p = jnp.pad(e2, pad12)

    hh = jnp.arange(16)[:, None]
    cc = jnp.arange(F)[None, :]
    sel = (cc // O == hh).astype(jnp.float32)
    tmat = ((jnp.arange(F)[:, None] % O) == jnp.arange(O)[None, :]).astype(
        jnp.float32) / H

    zm = jnp.zeros((NP, FJ), jnp.float32)

    ae1, ae2 = _ae(edge_attr, le1, e1p, le2, e2p)

    xwj1, xwq1 = _prep(x, w1, q1p, k1p)
    m1 = _edge_pass(src, dst, edge_type, ae1, xwj1, xwq1, zm)
    h = _comb1(m1, sel, b1.reshape(1, F))

    xwj2, xwq2 = _prep(h, w2, q2p, k2p)
    m2 = _edge_pass(src, dst, edge_type, ae2, xwj2, xwq2, zm)
    out = _comb2(m2, sel, tmat, b2.reshape(1, O))
    return out


# speedup vs baseline: 103.4931x; 2.1614x over previous
import functools
import jax
import jax.numpy as jnp
from jax import lax
from jax.experimental import pallas as pl
from jax.experimental.pallas import tpu as pltpu
from jax.experimental.pallas import tpu_sc as plsc

N = 10000
E = 320000
F = 128
R = 8
H = 4
O = 32
NB = 1000
EB = 2000
NCORE = 2
NSUB = 16
NW = NCORE * NSUB
EPW = E // NW
SUB = 80
NSUBC = 1
C = SUB * NSUBC
NCHUNK = EPW // C
FJ = F + 16
NP = 10240
RPS = NP // NSUB


def _prep_body(x_ref, w_ref, qp_ref, kp_ref, xw_ref, xwq_ref, xwk_ref):
    r = pl.program_id(1)
    xw = jnp.dot(x_ref[...], w_ref[r], preferred_element_type=jnp.float32)
    xw_ref[...] = xw
    xwq_ref[...] = jnp.dot(xw, qp_ref[...], preferred_element_type=jnp.float32)
    xwk_ref[...] = jnp.dot(xw, kp_ref[...], preferred_element_type=jnp.float32)


def _prep(x, w, qpad, kpad):
    return pl.pallas_call(
        _prep_body,
        grid=(N // NB, R),
        in_specs=[
            pl.BlockSpec((NB, F), lambda i, r: (i, 0)),
            pl.BlockSpec((R, F, F), lambda i, r: (0, 0, 0)),
            pl.BlockSpec((F, 16), lambda i, r: (0, 0)),
            pl.BlockSpec((F, 16), lambda i, r: (0, 0)),
        ],
        out_specs=[
            pl.BlockSpec((NB, F), lambda i, r: (r * (N // NB) + i, 0)),
            pl.BlockSpec((NB, 16), lambda i, r: (r * (N // NB) + i, 0)),
            pl.BlockSpec((NB, 16), lambda i, r: (r * (N // NB) + i, 0)),
        ],
        out_shape=[
            jax.ShapeDtypeStruct((R * N, F), jnp.float32),
            jax.ShapeDtypeStruct((R * N, 16), jnp.float32),
            jax.ShapeDtypeStruct((R * N, 16), jnp.float32),
        ],
    )(x, w, qpad, kpad)


def _ae_body(ea_ref, bd1_ref, bd2_ref, ae1_ref, ae2_ref):
    ea = ea_ref[...]
    ae1_ref[...] = jnp.dot(ea, bd1_ref[...], preferred_element_type=jnp.float32)
    ae2_ref[...] = jnp.dot(ea, bd2_ref[...], preferred_element_type=jnp.float32)


def _ae(ea_p, bd1, bd2):
    return pl.pallas_call(
        _ae_body,
        grid=(E // 8 // EB,),
        in_specs=[
            pl.BlockSpec((EB, 128), lambda i: (i, 0)),
            pl.BlockSpec((128, 128), lambda i: (0, 0)),
            pl.BlockSpec((128, 128), lambda i: (0, 0)),
        ],
        out_specs=[
            pl.BlockSpec((EB, 128), lambda i: (i, 0)),
            pl.BlockSpec((EB, 128), lambda i: (i, 0)),
        ],
        out_shape=[
            jax.ShapeDtypeStruct((E // 8, 128), jnp.float32),
            jax.ShapeDtypeStruct((E // 8, 128), jnp.float32),
        ],
    )(ea_p, bd1, bd2)


NPAIR = (NCHUNK - 1) // 2


def _edge_body(src_h, dst_h, rel_h, ae_h, xw_h, xwq_h, xwk_h, zm_h, zd_h,
               msum_h, dsum_h,
               srcv, dstv, relv, fiv, fjv, dsc, aev0, aev1, xwv,
               qv0, qv1, kv0, kv1, exv0, exv1,
               accm, accd,
               sl0, sl1, sg0, sg1, ss0, ss1):
    c = lax.axis_index("c")
    s = lax.axis_index("s")
    base = (c * NSUB + s) * EPW
    lsem = (sl0, sl1)
    gsem = (sg0, sg1)
    ssem = (ss0, ss1)
    aevs = (aev0, aev1)
    qvs = (qv0, qv1)
    kvs = (kv0, kv1)
    exvs = (exv0, exv1)

    pltpu.sync_copy(zm_h.at[pl.ds(s * RPS, RPS)], accm.at[pl.ds(s * RPS, RPS)])
    pltpu.sync_copy(zd_h.at[pl.ds(s * RPS, RPS)], accd.at[pl.ds(s * RPS, RPS)])
    plsc.subcore_barrier()

    def load(i, b):
        off = base + i * C
        return [pltpu.async_copy(src_h.at[pl.ds(off, C)], srcv.at[b], lsem[b]),
                pltpu.async_copy(dst_h.at[pl.ds(off, C)], dstv.at[b], lsem[b]),
                pltpu.async_copy(rel_h.at[pl.ds(off, C)], relv.at[b], lsem[b]),
                pltpu.async_copy(ae_h.at[pl.ds(off * 16, C * 16)], aevs[b],
                                 lsem[b])]

    def wait_load(i, b):
        off = base + i * C
        for sr, dr in ((src_h, srcv), (dst_h, dstv), (rel_h, relv)):
            pltpu.make_async_copy(sr.at[pl.ds(off, C)], dr.at[b], lsem[b]).wait()
        pltpu.make_async_copy(ae_h.at[pl.ds(off * 16, C * 16)], aevs[b],
                              lsem[b]).wait()

    def gathers(b):
        for g in range(C // 16):
            sl = pl.ds(g * 16, 16)
            r16 = relv[b, sl] * N
            fiv[b, sl] = r16 + dstv[b, sl]
            fjv[b, sl] = r16 + srcv[b, sl]
        return [pltpu.async_copy(xw_h.at[fjv.at[b]], xwv.at[b], gsem[b]),
                pltpu.async_copy(xwq_h.at[fiv.at[b]], qvs[b], gsem[b]),
                pltpu.async_copy(xwk_h.at[fjv.at[b]], kvs[b], gsem[b])]

    def wait_gathers(b):
        pltpu.make_async_copy(xw_h.at[fjv.at[b]], xwv.at[b], gsem[b]).wait()
        pltpu.make_async_copy(xwq_h.at[fiv.at[b]], qvs[b], gsem[b]).wait()
        pltpu.make_async_copy(xwk_h.at[fjv.at[b]], kvs[b], gsem[b]).wait()

    iota = lax.iota(jnp.int32, 16)
    hcols = [jnp.full((16,), h, jnp.int32) for h in range(H)]

    def compute(b):
        for g in range(C // 16):
            sl = pl.ds(g * 16, 16)
            dsc[b, sl] = dstv[b, sl]

        def cbody(g, cr):
            rb = g * 16
            row = iota + rb
            exs = []
            for h in range(H):
                a = (plsc.load_gather(qvs[b], [row, hcols[h]])
                     + plsc.load_gather(kvs[b], [row, hcols[h]])
                     + plsc.load_gather(aevs[b], [row * 16 + h]))
                a = jnp.maximum(a, 0.2 * a)
                ex = jnp.exp(a)
                plsc.store_scatter(exvs[b], [row, hcols[h]], ex)
                exs.append(ex)
            for e in range(16):
                r = rb + e
                for h in range(H):
                    sc = exs[h][e]
                    for t in range(O // 16):
                        sl = pl.ds(h * O + t * 16, 16)
                        xwv[b, r, sl] = xwv[b, r, sl] * sc
            return cr
        lax.fori_loop(0, C // 16, cbody, 0)

    def scatter(b):
        return [pltpu.async_copy(xwv.at[b], accm.at[dsc.at[b]], ssem[b],
                                 add=True),
                pltpu.async_copy(exvs[b], accd.at[dsc.at[b]], ssem[b],
                                 add=True)]

    def wait_scatter(b):
        pltpu.make_async_copy(xwv.at[b], accm.at[dsc.at[b]], ssem[b]).wait()
        pltpu.make_async_copy(exvs[b], accd.at[dsc.at[b]], ssem[b]).wait()

    for cp in load(0, 0):
        cp.wait()
    load(1, 1)
    gathers(0)

    def steady(p, carry):
        i = 2 * p
        wait_load(i + 1, 1)
        @pl.when(p > 0)
        def _():
            wait_scatter(1)
        gathers(1)
        wait_gathers(0)
        compute(0)
        scatter(0)
        @pl.when(p + 1 < NPAIR)
        def _():
            load(i + 2, 0)
        wait_gathers(1)
        compute(1)
        scatter(1)
        wait_scatter(0)
        @pl.when(p + 1 < NPAIR)
        def _():
            wait_load(i + 2, 0)
            gathers(0)
            load(i + 3, 1)
        return carry

    lax.fori_loop(0, NPAIR, steady, 0)

    wait_scatter(1)
    for cp in load(NCHUNK - 1, 0):
        cp.wait()
    for cp in gathers(0):
        cp.wait()
    compute(0)
    for cp in scatter(0):
        cp.wait()

    plsc.subcore_barrier()
    pltpu.sync_copy(accm.at[pl.ds(s * RPS, RPS)],
                    msum_h.at[c, pl.ds(s * RPS, RPS)])
    pltpu.sync_copy(accd.at[pl.ds(s * RPS, RPS)],
                    dsum_h.at[c, pl.ds(s * RPS, RPS)])


@functools.partial(
    pl.kernel,
    out_type=[
        jax.ShapeDtypeStruct((NCORE, NP, F), jnp.float32),
        jax.ShapeDtypeStruct((NCORE, NP, 16), jnp.float32),
    ],
    mesh=plsc.VectorSubcoreMesh(core_axis_name="c", subcore_axis_name="s"),
    scratch_types=[
        pltpu.VMEM((2, C), jnp.int32),
        pltpu.VMEM((2, C), jnp.int32),
        pltpu.VMEM((2, C), jnp.int32),
        pltpu.VMEM((2, C), jnp.int32),
        pltpu.VMEM((2, C), jnp.int32),
        pltpu.VMEM((2, C), jnp.int32),
        pltpu.VMEM((C * 16,), jnp.float32),
        pltpu.VMEM((C * 16,), jnp.float32),
        pltpu.VMEM((2, C, F), jnp.float32),
        pltpu.VMEM((C, 16), jnp.float32),
        pltpu.VMEM((C, 16), jnp.float32),
        pltpu.VMEM((C, 16), jnp.float32),
        pltpu.VMEM((C, 16), jnp.float32),
        pltpu.VMEM((C, 16), jnp.float32),
        pltpu.VMEM((C, 16), jnp.float32),
        pltpu.VMEM_SHARED((NP, F), jnp.float32),
        pltpu.VMEM_SHARED((NP, 16), jnp.float32),
        pltpu.SemaphoreType.DMA,
        pltpu.SemaphoreType.DMA,
        pltpu.SemaphoreType.DMA,
        pltpu.SemaphoreType.DMA,
        pltpu.SemaphoreType.DMA,
        pltpu.SemaphoreType.DMA,
    ],
    compiler_params=pltpu.CompilerParams(use_tc_tiling_on_sc=False,
                                         needs_layout_passes=False),
)
def _edge_pass(*args):
    _edge_body(*args)


def _comb1_body(m_ref, d_ref, sel_ref, b_ref, h_ref):
    M = m_ref[0] + m_ref[1]
    D = d_ref[0] + d_ref[1]
    dm = jnp.dot(D, sel_ref[...], preferred_element_type=jnp.float32)
    h_ref[...] = jnp.maximum(M / (dm + 1e-16) + b_ref[...], 0.0)


def _comb1(msum, dsum, sel, brow):
    return pl.pallas_call(
        _comb1_body,
        grid=(N // NB,),
        in_specs=[
            pl.BlockSpec((2, NB, F), lambda i: (0, i, 0)),
            pl.BlockSpec((2, NB, 16), lambda i: (0, i, 0)),
            pl.BlockSpec((16, F), lambda i: (0, 0)),
            pl.BlockSpec((1, F), lambda i: (0, 0)),
        ],
        out_specs=pl.BlockSpec((NB, F), lambda i: (i, 0)),
        out_shape=jax.ShapeDtypeStruct((N, F), jnp.float32),
    )(msum, dsum, sel, brow)


def _comb2_body(m_ref, d_ref, sel_ref, t_ref, b_ref, o_ref):
    M = m_ref[0] + m_ref[1]
    D = d_ref[0] + d_ref[1]
    dm = jnp.dot(D, sel_ref[...], preferred_element_type=jnp.float32)
    val = M / (dm + 1e-16)
    o_ref[...] = jnp.dot(val, t_ref[...], preferred_element_type=jnp.float32) + b_ref[...]


def _comb2(msum, dsum, sel, tmat, brow):
    return pl.pallas_call(
        _comb2_body,
        grid=(N // NB,),
        in_specs=[
            pl.BlockSpec((2, NB, F), lambda i: (0, i, 0)),
            pl.BlockSpec((2, NB, 16), lambda i: (0, i, 0)),
            pl.BlockSpec((16, F), lambda i: (0, 0)),
            pl.BlockSpec((F, O), lambda i: (0, 0)),
            pl.BlockSpec((1, O), lambda i: (0, 0)),
        ],
        out_specs=pl.BlockSpec((NB, O), lambda i: (i, 0)),
        out_shape=jax.ShapeDtypeStruct((N, O), jnp.float32),
    )(msum, dsum, sel, tmat, brow)


def kernel(x, edge_index, edge_attr, edge_type, w1, q1, k1, le1, e1, b1,
           w2, q2, k2, le2, e2, b2):
    src = edge_index[0]
    dst = edge_index[1]

    pad12 = ((0, 0), (0, 12))
    q1p = jnp.pad(q1, pad12)
    k1p = jnp.pad(k1, pad12)
    q2p = jnp.pad(q2, pad12)
    k2p = jnp.pad(k2, pad12)
    eye8 = jnp.eye(8, dtype=jnp.float32)
    bd1 = jnp.kron(eye8, jnp.pad(le1 @ e1, pad12))
    bd2 = jnp.kron(eye8, jnp.pad(le2 @ e2, pad12))
    ea_p = edge_attr.reshape(E // 8, 128)

    hh = jnp.arange(16)[:, None]
    cc = jnp.arange(F)[None, :]
    sel = (cc // O == hh).astype(jnp.float32)
    tmat = ((jnp.arange(F)[:, None] % O) == jnp.arange(O)[None, :]).astype(
        jnp.float32) / H

    zm = jnp.zeros((NP, F), jnp.float32)
    zd = jnp.zeros((NP, 16), jnp.float32)

    ae1_p, ae2_p = _ae(ea_p, bd1, bd2)
    ae1 = ae1_p.reshape(-1)
    ae2 = ae2_p.reshape(-1)

    xw1, xwq1, xwk1 = _prep(x, w1, q1p, k1p)
    m1, d1 = _edge_pass(src, dst, edge_type, ae1, xw1, xwq1, xwk1, zm, zd)
    h = _comb1(m1, d1, sel, b1.reshape(1, F))

    xw2, xwq2, xwk2 = _prep(h, w2, q2p, k2p)
    m2, d2 = _edge_pass(src, dst, edge_type, ae2, xw2, xwq2, xwk2, zm, zd)
    out = _comb2(m2, d2, sel, tmat, b2.reshape(1, O))
    return out
